# baseline (speedup 1.0000x reference)
import functools

import jax
import jax.numpy as jnp
from jax import lax
from jax.experimental import pallas as pl
from jax.experimental.pallas import tpu as pltpu
from jax.experimental.pallas import tpu_sc as plsc

L = 16
NC = 2
NS = 16
NW = NC * NS
ROWS = 128
COLS = 32768
RPW = ROWS // NW
CHUNKS = COLS // L
STREAMS = 4
CPS = CHUNKS // STREAMS
IMAX = 2**31 - 1


def _lane_shuffle(v, perm):
    return lax.gather(
        v, perm[:, None],
        lax.GatherDimensionNumbers(
            offset_dims=(), collapsed_slice_dims=(0,), start_index_map=(0,)),
        slice_sizes=(1,),
        mode=lax.GatherScatterMode.PROMISE_IN_BOUNDS)


def _row_argmax(buf, iota):
    ninf = jnp.full((L,), -jnp.inf, dtype=jnp.float32)
    zero = jnp.zeros((L,), dtype=jnp.int32)
    init = (ninf, ninf, ninf, ninf, zero, zero, zero, zero)

    def body(i, carry):
        maxs = list(carry[:STREAMS])
        cids = list(carry[STREAMS:])
        for s in range(STREAMS):
            c = i + s * CPS
            vals = buf[pl.ds(c * L, L)]
            pred = vals > maxs[s]
            maxs[s] = jnp.where(pred, vals, maxs[s])
            cids[s] = jnp.where(pred, lax.broadcast(c, (L,)), cids[s])
        return tuple(maxs) + tuple(cids)

    carry = lax.fori_loop(0, CPS, body, init, unroll=2)
    maxs = carry[:STREAMS]
    cids = carry[STREAMS:]
    m, cid = maxs[0], cids[0]
    for s in range(1, STREAMS):
        pred = maxs[s] > m
        m = jnp.where(pred, maxs[s], m)
        cid = jnp.where(pred, cids[s], cid)
    idx = cid * L + iota
    gmax = m
    for k in (1, 2, 4, 8):
        gmax = jnp.maximum(gmax, _lane_shuffle(gmax, iota ^ k))
    cand = jnp.where(m == gmax, idx, jnp.full((L,), IMAX, dtype=jnp.int32))
    for k in (1, 2, 4, 8):
        cand = jnp.minimum(cand, _lane_shuffle(cand, iota ^ k))
    return cand


def _argmax_kernel_body(x_hbm, out_hbm, buf0, buf1, outv, sem0, sem1):
    wid = lax.axis_index("s") * NC + lax.axis_index("c")
    base = wid * RPW
    bufs = [buf0, buf1]
    sems = [sem0, sem1]
    iota = lax.iota(jnp.int32, L)

    cps = [None] * RPW
    cps[0] = pltpu.async_copy(x_hbm.at[base], bufs[0], sems[0])
    res = jnp.zeros((L,), dtype=jnp.int32)
    for r in range(RPW):
        if r + 1 < RPW:
            cps[r + 1] = pltpu.async_copy(
                x_hbm.at[base + r + 1], bufs[(r + 1) % 2], sems[(r + 1) % 2])
        cps[r].wait()
        ans = _row_argmax(bufs[r % 2], iota)
        res = jnp.where(iota == r, ans, res)
    outv[...] = res
    pltpu.sync_copy(outv, out_hbm.at[wid])


@jax.jit
def kernel(x):
    mesh = plsc.VectorSubcoreMesh(core_axis_name="c", subcore_axis_name="s")
    out = pl.kernel(
        _argmax_kernel_body,
        mesh=mesh,
        out_type=jax.ShapeDtypeStruct((NW, L), jnp.int32),
        scratch_types=[
            pltpu.VMEM((COLS,), jnp.float32),
            pltpu.VMEM((COLS,), jnp.float32),
            pltpu.VMEM((L,), jnp.int32),
            pltpu.SemaphoreType.DMA,
            pltpu.SemaphoreType.DMA,
        ],
    )(x)
    return out[:, :RPW].reshape(ROWS).astype(jnp.int64)
